# baseline (speedup 1.0000x reference)
import functools

import jax
import jax.numpy as jnp
from jax import lax
from jax.experimental import pallas as pl
from jax.experimental.pallas import tpu as pltpu
from jax.experimental.pallas import tpu_sc as plsc

BATCH = 16384
COLS = 4
TOT = BATCH * COLS
NC, NS, L = 2, 16, 16
NW = NC * NS
PER_W = TOT // NW
VECS = PER_W // L

_DNUMS = lax.GatherDimensionNumbers(
    offset_dims=(), collapsed_slice_dims=(0,), start_index_map=(0,))


def _vgather(v, idx):
    return lax.gather(v, idx[:, None], dimension_numbers=_DNUMS,
                      slice_sizes=(1,),
                      mode=lax.GatherScatterMode.PROMISE_IN_BOUNDS)


_mesh = plsc.VectorSubcoreMesh(core_axis_name="c", subcore_axis_name="s")


@functools.partial(
    pl.kernel,
    mesh=_mesh,
    out_type=jax.ShapeDtypeStruct((TOT,), jnp.float32),
    scratch_types=[
        pltpu.VMEM((PER_W,), jnp.int32),
        pltpu.VMEM((PER_W,), jnp.float32),
        pltpu.VMEM((PER_W,), jnp.float32),
        pltpu.SemaphoreType.DMA,
    ],
)
def _bt_sc(xf, embed, out, idx_v, val_v, out_v, sem):
    wid = lax.axis_index("s") * NC + lax.axis_index("c")
    base = wid * PER_W
    pltpu.sync_copy(xf.at[pl.ds(base, PER_W)], idx_v)
    pltpu.async_copy(embed.at[idx_v], val_v, sem).wait()

    lane = lax.iota(jnp.int32, L)
    p1 = lane ^ 1
    p2 = lane ^ 2

    def body(i, carry):
        v = val_v[pl.ds(i * L, L)]
        a = v + _vgather(v, p1)
        rs = a + _vgather(a, p2)
        out_v[pl.ds(i * L, L)] = 4.0 * v - rs
        return carry

    lax.fori_loop(0, VECS, body, 0)
    pltpu.sync_copy(out_v, out.at[pl.ds(base, PER_W)])


def kernel(X, embed):
    xf = X.astype(jnp.int32).reshape(TOT)
    ef = embed.reshape(embed.shape[0])
    return _bt_sc(xf, ef).reshape(BATCH, COLS)
